# baseline (speedup 1.0000x reference)
import functools

import jax
import jax.numpy as jnp
from jax import lax
from jax.experimental import pallas as pl
from jax.experimental.pallas import tpu as pltpu
from jax.experimental.pallas import tpu_sc as plsc

N_NODES = 10000
SEQ_LEN = 32
IN_CH = 128
OUT_CH = 128

NUM_CORES = 2
NUM_SUBCORES = 16
NUM_WORKERS = NUM_CORES * NUM_SUBCORES

PIPE = 5
NODES_PER_PIPE = N_NODES // PIPE
CHUNK = 400
N_CHUNKS = NODES_PER_PIPE // CHUNK


def _make_sc_gather_body(pipe):
    node0 = pipe * NODES_PER_PIPE

    def _sc_gather_body(table_hbm, idx_hbm, out_hbm,
                        idx_v0, idx_v1, rows_v0, rows_v1,
                        gsem0, gsem1, wsem0, wsem1):
        wid = lax.axis_index("s") * NUM_CORES + lax.axis_index("c")
        idx_v = (idx_v0, idx_v1)
        rows_v = (rows_v0, rows_v1)
        gsem = (gsem0, gsem1)
        wsem = (wsem0, wsem1)

        def start_gather(c):
            b = c % 2
            pltpu.sync_copy(
                idx_hbm.at[pl.ds(wid * N_NODES + node0 + c * CHUNK, CHUNK)],
                idx_v[b])
            pltpu.make_async_copy(
                table_hbm.at[idx_v[b]], rows_v[b], gsem[b]).start()

        start_gather(0)
        start_gather(1)
        for c in range(N_CHUNKS):
            b = c % 2
            pltpu.make_async_copy(
                table_hbm.at[idx_v[b]], rows_v[b], gsem[b]).wait()
            wb = pltpu.make_async_copy(
                rows_v[b],
                out_hbm.at[pl.ds(wid * NODES_PER_PIPE + c * CHUNK, CHUNK)],
                wsem[b])
            wb.start()
            if c + 2 < N_CHUNKS:
                wb.wait()
                start_gather(c + 2)
            else:
                wb.wait()

    return _sc_gather_body


def _sc_gather(x, idx_flat, pipe):
    mesh = plsc.VectorSubcoreMesh(core_axis_name="c", subcore_axis_name="s")
    kfn = pl.kernel(
        _make_sc_gather_body(pipe),
        mesh=mesh,
        out_type=jax.ShapeDtypeStruct((SEQ_LEN * NODES_PER_PIPE, IN_CH),
                                      jnp.float32),
        scratch_types=[
            pltpu.VMEM((CHUNK,), jnp.int32),
            pltpu.VMEM((CHUNK,), jnp.int32),
            pltpu.VMEM((CHUNK, IN_CH), jnp.float32),
            pltpu.VMEM((CHUNK, IN_CH), jnp.float32),
            pltpu.SemaphoreType.DMA,
            pltpu.SemaphoreType.DMA,
            pltpu.SemaphoreType.DMA,
            pltpu.SemaphoreType.DMA,
        ],
    )
    return kfn(x, idx_flat)


def _mm_body(g_ref, wt_ref, b_ref, o_ref):
    acc = jnp.broadcast_to(b_ref[...], o_ref.shape)
    for s in range(SEQ_LEN):
        acc = acc + lax.dot_general(
            g_ref[s].astype(jnp.bfloat16), wt_ref[s].astype(jnp.bfloat16),
            (((1,), (0,)), ((), ())),
            preferred_element_type=jnp.float32,
        )
    o_ref[...] = acc


def _tc_matmul(gout, Wt, b):
    m_block = 1000
    grid = (NODES_PER_PIPE // m_block,)
    return pl.pallas_call(
        _mm_body,
        grid=grid,
        in_specs=[
            pl.BlockSpec((SEQ_LEN, m_block, IN_CH), lambda i: (0, i, 0)),
            pl.BlockSpec((SEQ_LEN, IN_CH, OUT_CH), lambda i: (0, 0, 0)),
            pl.BlockSpec((1, OUT_CH), lambda i: (0, 0)),
        ],
        out_specs=pl.BlockSpec((m_block, OUT_CH), lambda i: (i, 0)),
        out_shape=jax.ShapeDtypeStruct((NODES_PER_PIPE, OUT_CH), jnp.float32),
    )(gout, Wt, b)


@jax.jit
def kernel(x, indices, W, b):
    idx_flat = indices.astype(jnp.int32).T.reshape(-1)
    Wt = W.reshape(OUT_CH, SEQ_LEN, IN_CH).transpose(1, 2, 0)
    b2 = b.reshape(1, OUT_CH)
    outs = []
    for p in range(PIPE):
        g = _sc_gather(x, idx_flat, p)
        gout = g.reshape(SEQ_LEN, NODES_PER_PIPE, IN_CH)
        outs.append(_tc_matmul(gout, Wt, b2))
    return jnp.concatenate(outs, axis=0)


# speedup vs baseline: 1.0179x; 1.0179x over previous
import functools

import jax
import jax.numpy as jnp
from jax import lax
from jax.experimental import pallas as pl
from jax.experimental.pallas import tpu as pltpu
from jax.experimental.pallas import tpu_sc as plsc

N_NODES = 10000
SEQ_LEN = 32
IN_CH = 128
OUT_CH = 128

NUM_CORES = 2
NUM_SUBCORES = 16
NUM_WORKERS = NUM_CORES * NUM_SUBCORES

PIPE = 2
NODES_PER_PIPE = N_NODES // PIPE
CHUNK = 200
N_CHUNKS = NODES_PER_PIPE // CHUNK


def _make_sc_gather_body(pipe):
    node0 = pipe * NODES_PER_PIPE

    def _sc_gather_body(table_hbm, idx_hbm, out_hbm,
                        idx_all, rows_v0, rows_v1,
                        gsem0, gsem1, wsem0, wsem1):
        wid = lax.axis_index("s") * NUM_CORES + lax.axis_index("c")
        rows_v = (rows_v0, rows_v1)
        gsem = (gsem0, gsem1)
        wsem = (wsem0, wsem1)

        pltpu.sync_copy(
            idx_hbm.at[pl.ds(wid * N_NODES + node0, NODES_PER_PIPE)], idx_all)

        def start_gather(c):
            b = c % 2
            pltpu.make_async_copy(
                table_hbm.at[idx_all.at[pl.ds(c * CHUNK, CHUNK)]],
                rows_v[b], gsem[b]).start()

        start_gather(0)
        start_gather(1)
        for c in range(N_CHUNKS):
            b = c % 2
            pltpu.make_async_copy(
                table_hbm.at[idx_all.at[pl.ds(c * CHUNK, CHUNK)]],
                rows_v[b], gsem[b]).wait()
            wb = pltpu.make_async_copy(
                rows_v[b],
                out_hbm.at[pl.ds(wid * NODES_PER_PIPE + c * CHUNK, CHUNK)],
                wsem[b])
            wb.start()
            if c + 2 < N_CHUNKS:
                wb.wait()
                start_gather(c + 2)
            else:
                wb.wait()

    return _sc_gather_body


def _sc_gather(x, idx_flat, pipe):
    mesh = plsc.VectorSubcoreMesh(core_axis_name="c", subcore_axis_name="s")
    kfn = pl.kernel(
        _make_sc_gather_body(pipe),
        mesh=mesh,
        out_type=jax.ShapeDtypeStruct((SEQ_LEN * NODES_PER_PIPE, IN_CH),
                                      jnp.float32),
        scratch_types=[
            pltpu.VMEM((NODES_PER_PIPE,), jnp.int32),
            pltpu.VMEM((CHUNK, IN_CH), jnp.float32),
            pltpu.VMEM((CHUNK, IN_CH), jnp.float32),
            pltpu.SemaphoreType.DMA,
            pltpu.SemaphoreType.DMA,
            pltpu.SemaphoreType.DMA,
            pltpu.SemaphoreType.DMA,
        ],
    )
    return kfn(x, idx_flat)


def _mm_body(g_ref, wt_ref, b_ref, o_ref):
    acc = jnp.broadcast_to(b_ref[...], o_ref.shape)
    for s in range(SEQ_LEN):
        acc = acc + lax.dot_general(
            g_ref[s].astype(jnp.bfloat16), wt_ref[s].astype(jnp.bfloat16),
            (((1,), (0,)), ((), ())),
            preferred_element_type=jnp.float32,
        )
    o_ref[...] = acc


def _tc_matmul(gout, Wt, b):
    m_block = 1000
    grid = (NODES_PER_PIPE // m_block,)
    return pl.pallas_call(
        _mm_body,
        grid=grid,
        in_specs=[
            pl.BlockSpec((SEQ_LEN, m_block, IN_CH), lambda i: (0, i, 0)),
            pl.BlockSpec((SEQ_LEN, IN_CH, OUT_CH), lambda i: (0, 0, 0)),
            pl.BlockSpec((1, OUT_CH), lambda i: (0, 0)),
        ],
        out_specs=pl.BlockSpec((m_block, OUT_CH), lambda i: (i, 0)),
        out_shape=jax.ShapeDtypeStruct((NODES_PER_PIPE, OUT_CH), jnp.float32),
    )(gout, Wt, b)


@jax.jit
def kernel(x, indices, W, b):
    idx_flat = indices.astype(jnp.int32).T.reshape(-1)
    Wt = W.reshape(OUT_CH, SEQ_LEN, IN_CH).transpose(1, 2, 0)
    b2 = b.reshape(1, OUT_CH)
    outs = []
    for p in range(PIPE):
        g = _sc_gather(x, idx_flat, p)
        gout = g.reshape(SEQ_LEN, NODES_PER_PIPE, IN_CH)
        outs.append(_tc_matmul(gout, Wt, b2))
    return jnp.concatenate(outs, axis=0)


# speedup vs baseline: 1.0364x; 1.0181x over previous
import functools

import jax
import jax.numpy as jnp
from jax import lax
from jax.experimental import pallas as pl
from jax.experimental.pallas import tpu as pltpu
from jax.experimental.pallas import tpu_sc as plsc

N_NODES = 10000
SEQ_LEN = 32
IN_CH = 128
OUT_CH = 128

NUM_CORES = 2
NUM_SUBCORES = 16
NUM_WORKERS = NUM_CORES * NUM_SUBCORES
ROWS_PER_WORKER = N_NODES

CHUNK = 200
N_CHUNKS = ROWS_PER_WORKER // CHUNK
NBUF = 4


def _sc_gather_body(table_hbm, idx_hbm, out_hbm, idx_all, *bufs):
    rows_v = bufs[:NBUF]
    gsem = bufs[NBUF:2 * NBUF]
    wsem = bufs[2 * NBUF:3 * NBUF]
    wid = lax.axis_index("s") * NUM_CORES + lax.axis_index("c")
    base = wid * ROWS_PER_WORKER

    pltpu.sync_copy(idx_hbm.at[pl.ds(base, ROWS_PER_WORKER)], idx_all)

    def start_gather(c):
        b = c % NBUF
        pltpu.make_async_copy(
            table_hbm.at[idx_all.at[pl.ds(c * CHUNK, CHUNK)]],
            rows_v[b], gsem[b]).start()

    for c in range(NBUF):
        start_gather(c)
    for c in range(N_CHUNKS):
        b = c % NBUF
        pltpu.make_async_copy(
            table_hbm.at[idx_all.at[pl.ds(c * CHUNK, CHUNK)]],
            rows_v[b], gsem[b]).wait()
        wb = pltpu.make_async_copy(
            rows_v[b], out_hbm.at[pl.ds(base + c * CHUNK, CHUNK)], wsem[b])
        wb.start()
        if c + NBUF < N_CHUNKS:
            wb.wait()
            start_gather(c + NBUF)
        else:
            wb.wait()


def _sc_gather(x, idx_flat):
    mesh = plsc.VectorSubcoreMesh(core_axis_name="c", subcore_axis_name="s")
    kfn = pl.kernel(
        _sc_gather_body,
        mesh=mesh,
        out_type=jax.ShapeDtypeStruct((SEQ_LEN * N_NODES, IN_CH), jnp.float32),
        scratch_types=(
            [pltpu.VMEM((ROWS_PER_WORKER,), jnp.int32)]
            + [pltpu.VMEM((CHUNK, IN_CH), jnp.float32)] * NBUF
            + [pltpu.SemaphoreType.DMA] * (2 * NBUF)
        ),
    )
    return kfn(x, idx_flat)


def _mm_body(g_ref, wt_ref, b_ref, o_ref):
    acc = jnp.broadcast_to(b_ref[...], o_ref.shape)
    for s in range(SEQ_LEN):
        acc = acc + lax.dot_general(
            g_ref[s].astype(jnp.bfloat16), wt_ref[s].astype(jnp.bfloat16),
            (((1,), (0,)), ((), ())),
            preferred_element_type=jnp.float32,
        )
    o_ref[...] = acc


def _tc_matmul(gout, Wt, b):
    m_block = 1000
    grid = (N_NODES // m_block,)
    return pl.pallas_call(
        _mm_body,
        grid=grid,
        in_specs=[
            pl.BlockSpec((SEQ_LEN, m_block, IN_CH), lambda i: (0, i, 0)),
            pl.BlockSpec((SEQ_LEN, IN_CH, OUT_CH), lambda i: (0, 0, 0)),
            pl.BlockSpec((1, OUT_CH), lambda i: (0, 0)),
        ],
        out_specs=pl.BlockSpec((m_block, OUT_CH), lambda i: (i, 0)),
        out_shape=jax.ShapeDtypeStruct((N_NODES, OUT_CH), jnp.float32),
    )(gout, Wt, b)


@jax.jit
def kernel(x, indices, W, b):
    idx_flat = indices.astype(jnp.int32).T.reshape(-1)
    Wt = W.reshape(OUT_CH, SEQ_LEN, IN_CH).transpose(1, 2, 0)
    g = _sc_gather(x, idx_flat)
    gout = g.reshape(SEQ_LEN, N_NODES, IN_CH)
    return _tc_matmul(gout, Wt, b.reshape(1, OUT_CH))
